# baseline (device time: 49643 ns/iter reference)
import jax
import jax.numpy as jnp
from jax import lax
from jax.experimental import pallas as pl
from jax.experimental.pallas import tpu as pltpu

NC = 8


def kernel(x, dy):
    k_per, d = x.shape
    _, f = dy.shape
    half = d // 2
    fhalf = f // 2
    fc = fhalf // NC

    def body(
        x_ref, dy_ref, out_ref,
        send_x, recv_x, send_y, recv_y,
        sx_sems, rx_sems, sy_sems, ry_sems,
    ):
        ix = lax.axis_index("x")
        iy = lax.axis_index("y")
        iz = lax.axis_index("z")
        px = 1 - ix
        py = iy ^ 1
        h = iy & 1

        barrier_sem = pltpu.get_barrier_semaphore()
        pl.semaphore_signal(
            barrier_sem, inc=1,
            device_id=(px, iy, iz), device_id_type=pl.DeviceIdType.MESH,
        )
        pl.semaphore_signal(
            barrier_sem, inc=1,
            device_id=(ix, py, iz), device_id_type=pl.DeviceIdType.MESH,
        )
        pl.semaphore_wait(barrier_sem, 2)

        x_p = x_ref[:, pl.ds(px * half, half)].astype(jnp.bfloat16)
        x_m = x_ref[:, pl.ds(ix * half, half)].astype(jnp.bfloat16)

        dims = (((0,), (0,)), ((), ()))
        my_base = h * fhalf
        other_base = (1 - h) * fhalf

        x_rdmas = []
        for j in range(NC):
            mycols = pl.ds(my_base + j * fc, fc)
            dyb_j = dy_ref[:, mycols].astype(jnp.bfloat16)
            pp_j = lax.dot_general(
                x_p, dyb_j, dims, preferred_element_type=jnp.float32
            )
            send_x[j] = pp_j.astype(jnp.bfloat16)
            rdma = pltpu.make_async_remote_copy(
                src_ref=send_x.at[j],
                dst_ref=recv_x.at[j],
                send_sem=sx_sems.at[j],
                recv_sem=rx_sems.at[j],
                device_id=(px, iy, iz),
                device_id_type=pl.DeviceIdType.MESH,
            )
            rdma.start()
            x_rdmas.append(rdma)
            pm_j = lax.dot_general(
                x_m, dyb_j, dims, preferred_element_type=jnp.float32
            )
            out_ref[:, mycols] = pm_j

        y_rdmas = []
        for j in range(NC):
            mycols = pl.ds(my_base + j * fc, fc)
            x_rdmas[j].wait()
            r_j = out_ref[:, mycols] + recv_x[j].astype(jnp.float32)
            out_ref[:, mycols] = r_j
            send_y[j] = r_j.astype(jnp.bfloat16)
            rdma = pltpu.make_async_remote_copy(
                src_ref=send_y.at[j],
                dst_ref=recv_y.at[j],
                send_sem=sy_sems.at[j],
                recv_sem=ry_sems.at[j],
                device_id=(ix, py, iz),
                device_id_type=pl.DeviceIdType.MESH,
            )
            rdma.start()
            y_rdmas.append(rdma)

        for j in range(NC):
            othercols = pl.ds(other_base + j * fc, fc)
            y_rdmas[j].wait()
            out_ref[:, othercols] = recv_y[j].astype(jnp.float32)

    buf = pltpu.VMEM((NC, half, fc), jnp.bfloat16)
    return pl.pallas_call(
        body,
        out_shape=jax.ShapeDtypeStruct((half, f), jnp.float32),
        in_specs=[
            pl.BlockSpec(memory_space=pltpu.VMEM),
            pl.BlockSpec(memory_space=pltpu.VMEM),
        ],
        out_specs=pl.BlockSpec(memory_space=pltpu.VMEM),
        scratch_shapes=[
            buf, buf, buf, buf,
            pltpu.SemaphoreType.DMA((NC,)),
            pltpu.SemaphoreType.DMA((NC,)),
            pltpu.SemaphoreType.DMA((NC,)),
            pltpu.SemaphoreType.DMA((NC,)),
        ],
        compiler_params=pltpu.CompilerParams(
            collective_id=0, vmem_limit_bytes=100 * 1024 * 1024
        ),
    )(x, dy)
